# baseline (speedup 1.0000x reference)
import functools

import jax
import jax.numpy as jnp
import numpy as np
from jax import lax
from jax.experimental import pallas as pl
from jax.experimental.pallas import tpu as pltpu
from jax.experimental.pallas import tpu_sc as plsc

_BN_EPS = 1e-5
_NC = 2
_NS = 16
_NW = _NC * _NS
_K = 80


def _fold(p):
  W, b, g, be = p["W"], p["b"], p["g"], p["be"]
  ci = W.shape[0] // 2
  s = (g / np.sqrt(1.0 + _BN_EPS)).astype(jnp.float32)
  wa = (W[:ci] - W[ci:]) * s[None, :]
  wb = W[ci:] * s[None, :]
  c = b * s + be
  return wa, wb, c.reshape(1, -1)


def _edge_pass(a, b, dst3, src3, with_deg, ab2=None):
  n_nodes, co = a.shape
  nch = dst3.shape[1]
  npt = n_nodes // _NS
  nzc = 5
  nzr = npt // nzc
  cg = co // 16
  mesh = plsc.VectorSubcoreMesh(core_axis_name="c", subcore_axis_name="s")

  out_type = [jax.ShapeDtypeStruct((_NC, n_nodes, co), jnp.float32)]
  if ab2 is not None:
    out_type.append(jax.ShapeDtypeStruct((_NC, n_nodes, co), jnp.float32))
  scratch = [
      pltpu.VMEM((nch, _K), jnp.int32),
      pltpu.VMEM((nch, _K), jnp.int32),
      pltpu.VMEM((_K, co), jnp.float32),
      pltpu.VMEM((_K, co), jnp.float32),
      pltpu.VMEM((_K, co), jnp.float32),
      pltpu.VMEM((_K, co), jnp.float32),
      pltpu.VMEM((_K, co), jnp.float32),
      pltpu.VMEM((_K, co), jnp.float32),
      pltpu.VMEM((nzr, co), jnp.float32),
      pltpu.VMEM_SHARED((n_nodes, co), jnp.float32),
      pltpu.SemaphoreType.DMA,
      pltpu.SemaphoreType.DMA,
      pltpu.SemaphoreType.DMA,
      pltpu.SemaphoreType.DMA,
  ]
  if with_deg:
    out_type.append(jax.ShapeDtypeStruct((_NW, n_nodes), jnp.float32))
    scratch.append(pltpu.VMEM((n_nodes,), jnp.float32))

  @functools.partial(
      pl.kernel,
      out_type=tuple(out_type) if len(out_type) > 1 else out_type[0],
      mesh=mesh,
      compiler_params=pltpu.CompilerParams(
          use_tc_tiling_on_sc=False, needs_layout_passes=False),
      scratch_types=scratch,
  )
  def ek(*refs):
    n_in = 4 if ab2 is None else 6
    n_out = len(out_type)
    ins, outs, rest = refs[:n_in], refs[n_in:n_in + n_out], refs[n_in + n_out:]
    dst_hbm, src_hbm = ins[n_in - 2], ins[n_in - 1]
    if with_deg:
      (dstv, srcv, bufa0, bufb0, bufa1, bufb1, bufm0, bufm1,
       zbuf, acc, sem0, sem1, ssc0, ssc1, degacc) = rest
    else:
      (dstv, srcv, bufa0, bufb0, bufa1, bufb1, bufm0, bufm1,
       zbuf, acc, sem0, sem1, ssc0, ssc1) = rest
    cid = lax.axis_index("c")
    sid = lax.axis_index("s")
    wid = sid * _NC + cid
    row0 = sid * npt
    pltpu.sync_copy(dst_hbm.at[wid], dstv)
    pltpu.sync_copy(src_hbm.at[wid], srcv)

    z16 = jnp.zeros((16,), jnp.float32)

    def zero_acc():
      def zrow(i, _):
        def zcol(j, _):
          zbuf[i, pl.ds(j * 16, 16)] = z16
          return 0
        return lax.fori_loop(0, cg, zcol, 0)
      lax.fori_loop(0, nzr, zrow, 0)
      for q in range(nzc):
        pltpu.sync_copy(zbuf, acc.at[pl.ds(row0 + q * nzr, nzr)])

    zero_acc()

    if with_deg:
      deg_hbm = outs[1]

      def zdeg(i, _):
        degacc[pl.ds(i * 16, 16)] = z16
        return 0
      lax.fori_loop(0, n_nodes // 16, zdeg, 0)
      one16 = jnp.ones((16,), jnp.float32)

      def drow(kc, _):
        for gq in range(_K // 16):
          idx = dstv[kc, pl.ds(gq * 16, 16)]
          plsc.addupdate_scatter(degacc, [idx], one16)
        return 0
      lax.fori_loop(0, nch, drow, 0)
      pltpu.sync_copy(degacc, deg_hbm.at[wid])

    def run_pass(a_hbm, b_hbm, out_hbm):
      plsc.subcore_barrier()

      def issue(kc, ba, bb, sem):
        pltpu.async_copy(a_hbm.at[dstv.at[kc]], ba, sem)
        pltpu.async_copy(b_hbm.at[srcv.at[kc]], bb, sem)

      def wait(ba, bb, sem):
        pltpu.make_async_copy(a_hbm.at[dstv.at[0]], ba, sem).wait()
        pltpu.make_async_copy(b_hbm.at[srcv.at[0]], bb, sem).wait()

      def compute(ba, bb, bm):
        def erow(t, _):
          for u in range(4):
            i = t * 4 + u
            for j in range(cg):
              d = pl.ds(j * 16, 16)
              bm[i, d] = jnp.maximum(ba[i, d] + bb[i, d], 0.0)
          return 0
        lax.fori_loop(0, _K // 4, erow, 0)

      def iscatter(kc, bm, sem):
        pltpu.async_copy(bm, acc.at[dstv.at[kc]], sem, add=True)

      def wscatter(bm, sem):
        pltpu.make_async_copy(bm, acc.at[dstv.at[0]], sem).wait()

      issue(0, bufa0, bufb0, sem0)
      issue(1, bufa1, bufb1, sem1)
      wait(bufa0, bufb0, sem0)
      compute(bufa0, bufb0, bufm0)
      iscatter(0, bufm0, ssc0)
      issue(2, bufa0, bufb0, sem0)
      wait(bufa1, bufb1, sem1)
      compute(bufa1, bufb1, bufm1)
      iscatter(1, bufm1, ssc1)

      def pair(p, _):
        kc0 = 2 * p
        issue(kc0 + 1, bufa1, bufb1, sem1)
        wait(bufa0, bufb0, sem0)
        wscatter(bufm0, ssc0)
        compute(bufa0, bufb0, bufm0)
        iscatter(kc0, bufm0, ssc0)
        issue(kc0 + 2, bufa0, bufb0, sem0)
        wait(bufa1, bufb1, sem1)
        wscatter(bufm1, ssc1)
        compute(bufa1, bufb1, bufm1)
        iscatter(kc0 + 1, bufm1, ssc1)
        return 0
      lax.fori_loop(1, (nch - 1) // 2, pair, 0)
      wait(bufa0, bufb0, sem0)
      wscatter(bufm0, ssc0)
      compute(bufa0, bufb0, bufm0)
      iscatter(nch - 1, bufm0, ssc0)
      wscatter(bufm0, ssc0)
      wscatter(bufm1, ssc1)

      plsc.subcore_barrier()
      pltpu.sync_copy(acc.at[pl.ds(row0, npt)],
                      out_hbm.at[cid, pl.ds(row0, npt)])

    run_pass(ins[0], ins[1], outs[0])
    if ab2 is not None:
      zero_acc()
      run_pass(ins[2], ins[3], outs[1])

  if ab2 is None:
    return ek(a, b, dst3, src3)
  return ek(a, b, ab2[0], ab2[1], dst3, src3)


def _pool_pass(s3, inv3, bp3, g4):
  npn = bp3.shape[2]
  mesh = plsc.VectorSubcoreMesh(core_axis_name="c", subcore_axis_name="s")

  @functools.partial(
      pl.kernel,
      out_type=jax.ShapeDtypeStruct((_NW, 3, g4, 16), jnp.float32),
      mesh=mesh,
      compiler_params=pltpu.CompilerParams(
          use_tc_tiling_on_sc=False, needs_layout_passes=False),
      scratch_types=[
          pltpu.VMEM((npn, 16), jnp.float32),
          pltpu.VMEM((npn, 16), jnp.float32),
          pltpu.VMEM((1, npn), jnp.float32),
          pltpu.VMEM((1, npn), jnp.int32),
          pltpu.VMEM((g4, 16), jnp.float32),
          pltpu.VMEM((g4, 16), jnp.float32),
          pltpu.VMEM((g4, 16), jnp.float32),
      ],
  )
  def pk(s_hbm, inv_hbm, b_hbm, out_hbm, s0buf, s1buf, ibuf, bbuf,
         sacc, cacc, macc):
    cid = lax.axis_index("c")
    sid = lax.axis_index("s")
    wid = sid * _NC + cid
    w0 = wid * npn
    pltpu.sync_copy(s_hbm.at[0, pl.ds(w0, npn)], s0buf)
    pltpu.sync_copy(s_hbm.at[1, pl.ds(w0, npn)], s1buf)
    pltpu.sync_copy(inv_hbm.at[wid], ibuf)
    pltpu.sync_copy(b_hbm.at[wid], bbuf)

    z16 = jnp.zeros((16,), jnp.float32)
    ninf = jnp.full((16,), -jnp.inf, jnp.float32)

    def init(i, _):
      sacc[i, :] = z16
      cacc[i, :] = z16
      macc[i, :] = ninf
      return 0
    lax.fori_loop(0, g4, init, 0)

    col = lax.iota(jnp.int32, 16)
    one16 = jnp.ones((16,), jnp.float32)

    def node16(t, _):
      bidx = bbuf[0, pl.ds(t * 16, 16)]
      invs = ibuf[0, pl.ds(t * 16, 16)]
      for lane in range(16):
        bb = jnp.full((16,), bidx[lane], jnp.int32)
        i = t * 16 + lane
        hrow = jnp.maximum((s0buf[i, :] + s1buf[i, :]) * invs[lane], 0.0)
        s = plsc.load_gather(sacc, [bb, col])
        plsc.store_scatter(sacc, [bb, col], s + hrow)
        c = plsc.load_gather(cacc, [bb, col])
        plsc.store_scatter(cacc, [bb, col], c + one16)
        m = plsc.load_gather(macc, [bb, col])
        plsc.store_scatter(macc, [bb, col], jnp.maximum(m, hrow))
      return 0
    lax.fori_loop(0, npn // 16, node16, 0)

    pltpu.sync_copy(sacc, out_hbm.at[wid, 0])
    pltpu.sync_copy(cacc, out_hbm.at[wid, 1])
    pltpu.sync_copy(macc, out_hbm.at[wid, 2])

  return pk(s3, inv3, bp3)


_BLK = 10240


def _tc_pre0(x, wa, wb, c):
  n, ci = x.shape
  co = wa.shape[1]

  def body(x_ref, wa_ref, wb_ref, c_ref, a_ref, b_ref):
    h = x_ref[...]
    a_ref[...] = jnp.dot(h, wa_ref[...],
                         preferred_element_type=jnp.float32) + c_ref[...]
    b_ref[...] = jnp.dot(h, wb_ref[...], preferred_element_type=jnp.float32)

  return pl.pallas_call(
      body,
      grid=(n // _BLK,),
      in_specs=[
          pl.BlockSpec((_BLK, ci), lambda i: (i, 0)),
          pl.BlockSpec((ci, co), lambda i: (0, 0)),
          pl.BlockSpec((ci, co), lambda i: (0, 0)),
          pl.BlockSpec((1, co), lambda i: (0, 0)),
      ],
      out_specs=[pl.BlockSpec((_BLK, co), lambda i: (i, 0)),
                 pl.BlockSpec((_BLK, co), lambda i: (i, 0))],
      out_shape=[jax.ShapeDtypeStruct((n, co), jnp.float32)] * 2,
  )(x, wa, wb, c)


def _tc_postpre_deg(s, degp, wa, wb, c):
  n = s.shape[1]
  cp, co = wa.shape

  def body(s_ref, deg_ref, wa_ref, wb_ref, c_ref, a_ref, b_ref, inv_ref):
    st = s_ref[0] + s_ref[1]
    deg = jnp.sum(deg_ref[...], axis=1, keepdims=True)
    invd = 1.0 / jnp.maximum(deg, 1.0)
    h = jnp.maximum(st * invd, 0.0)
    a_ref[...] = jnp.dot(h, wa_ref[...],
                         preferred_element_type=jnp.float32) + c_ref[...]
    b_ref[...] = jnp.dot(h, wb_ref[...], preferred_element_type=jnp.float32)
    inv_ref[...] = invd

  return pl.pallas_call(
      body,
      grid=(n // _BLK,),
      in_specs=[
          pl.BlockSpec((2, _BLK, cp), lambda i: (0, i, 0)),
          pl.BlockSpec((_BLK, _NW), lambda i: (i, 0)),
          pl.BlockSpec((cp, co), lambda i: (0, 0)),
          pl.BlockSpec((cp, co), lambda i: (0, 0)),
          pl.BlockSpec((1, co), lambda i: (0, 0)),
      ],
      out_specs=[pl.BlockSpec((_BLK, co), lambda i: (i, 0)),
                 pl.BlockSpec((_BLK, co), lambda i: (i, 0)),
                 pl.BlockSpec((_BLK, 1), lambda i: (i, 0))],
      out_shape=[jax.ShapeDtypeStruct((n, co), jnp.float32),
                 jax.ShapeDtypeStruct((n, co), jnp.float32),
                 jax.ShapeDtypeStruct((n, 1), jnp.float32)],
  )(s, degp, wa, wb, c)


def _tc_postpre(s, invd, wa, wb, c):
  n, cp = s.shape[1], s.shape[2]
  co = wa.shape[1]

  def body(s_ref, inv_ref, wa_ref, wb_ref, c_ref, a_ref, b_ref):
    h = jnp.maximum((s_ref[0] + s_ref[1]) * inv_ref[...], 0.0)
    a_ref[...] = jnp.dot(h, wa_ref[...],
                         preferred_element_type=jnp.float32) + c_ref[...]
    b_ref[...] = jnp.dot(h, wb_ref[...], preferred_element_type=jnp.float32)

  return pl.pallas_call(
      body,
      grid=(n // _BLK,),
      in_specs=[
          pl.BlockSpec((2, _BLK, cp), lambda i: (0, i, 0)),
          pl.BlockSpec((_BLK, 1), lambda i: (i, 0)),
          pl.BlockSpec((cp, co), lambda i: (0, 0)),
          pl.BlockSpec((cp, co), lambda i: (0, 0)),
          pl.BlockSpec((1, co), lambda i: (0, 0)),
      ],
      out_specs=[pl.BlockSpec((_BLK, co), lambda i: (i, 0)),
                 pl.BlockSpec((_BLK, co), lambda i: (i, 0))],
      out_shape=[jax.ShapeDtypeStruct((n, co), jnp.float32)] * 2,
  )(s, invd, wa, wb, c)


def _tc_post2(sl, sr, invd):
  n, cp = sl.shape[1], sl.shape[2]

  def body(sl_ref, sr_ref, inv_ref, o_ref):
    hl = jnp.maximum((sl_ref[0] + sl_ref[1]) * inv_ref[...], 0.0)
    hr = jnp.maximum((sr_ref[0] + sr_ref[1]) * inv_ref[...], 0.0)
    o_ref[...] = jnp.concatenate([hl, hr], axis=1)

  return pl.pallas_call(
      body,
      grid=(n // _BLK,),
      in_specs=[
          pl.BlockSpec((2, _BLK, cp), lambda i: (0, i, 0)),
          pl.BlockSpec((2, _BLK, cp), lambda i: (0, i, 0)),
          pl.BlockSpec((_BLK, 1), lambda i: (i, 0)),
      ],
      out_specs=pl.BlockSpec((_BLK, 2 * cp), lambda i: (i, 0)),
      out_shape=jax.ShapeDtypeStruct((n, 2 * cp), jnp.float32),
  )(sl, sr, invd)


def _tc_post(s, invd):
  n, cp = s.shape[1], s.shape[2]

  def body(s_ref, inv_ref, o_ref):
    o_ref[...] = jnp.maximum((s_ref[0] + s_ref[1]) * inv_ref[...], 0.0)

  return pl.pallas_call(
      body,
      grid=(n // _BLK,),
      in_specs=[
          pl.BlockSpec((2, _BLK, cp), lambda i: (0, i, 0)),
          pl.BlockSpec((_BLK, 1), lambda i: (i, 0)),
      ],
      out_specs=pl.BlockSpec((_BLK, cp), lambda i: (i, 0)),
      out_shape=jax.ShapeDtypeStruct((n, cp), jnp.float32),
  )(s, invd)


def _tc_mid(pool, w1a, w1b, b1, w2, b2, wd1, bd1, wd2, bd2, wa0, wb0, c0):
  g4 = pool.shape[2]
  co0 = wa0.shape[1]

  def body(p_ref, w1a_r, w1b_r, b1_r, w2_r, b2_r, wd1_r, bd1_r, wd2_r,
           bd2_r, wa0_r, wb0_r, c0_r, z_ref, a0_ref, b0_ref):
    def step(t, carry):
      su, cn, mx = carry
      return (su + p_ref[t, 0], cn + p_ref[t, 1],
              jnp.maximum(mx, p_ref[t, 2]))
    su, cn, mx = lax.fori_loop(
        0, _NW, step,
        (jnp.zeros((g4, 16), jnp.float32), jnp.zeros((g4, 16), jnp.float32),
         jnp.full((g4, 16), -jnp.inf, jnp.float32)))
    xm = su / jnp.maximum(cn, 1.0)
    g = jnp.maximum(
        jnp.dot(xm, w1a_r[...], preferred_element_type=jnp.float32)
        + jnp.dot(mx, w1b_r[...], preferred_element_type=jnp.float32)
        + b1_r[...], 0.0)
    z = jnp.maximum(
        jnp.dot(g, w2_r[...], preferred_element_type=jnp.float32)
        + b2_r[...], 0.0)
    z_ref[...] = z
    y = jnp.maximum(
        jnp.dot(z, wd1_r[...], preferred_element_type=jnp.float32)
        + bd1_r[...], 0.0)
    y = jnp.maximum(
        jnp.dot(y, wd2_r[...], preferred_element_type=jnp.float32)
        + bd2_r[...], 0.0)
    a0_ref[...] = jnp.dot(y, wa0_r[...],
                          preferred_element_type=jnp.float32) + c0_r[...]
    b0_ref[...] = jnp.dot(y, wb0_r[...], preferred_element_type=jnp.float32)

  return pl.pallas_call(
      body,
      out_shape=[jax.ShapeDtypeStruct((g4, 16), jnp.float32),
                 jax.ShapeDtypeStruct((g4, co0), jnp.float32),
                 jax.ShapeDtypeStruct((g4, co0), jnp.float32)],
  )(pool, w1a, w1b, b1, w2, b2, wd1, bd1, wd2, bd2, wa0, wb0, c0)


def kernel(x, params, edge_index, batch_index):
  n = x.shape[0]
  e = edge_index.shape[1]
  g = 100
  g4 = 104
  npd = _NW * 320
  nch = e // (_NW * _K)
  src3 = edge_index[0].reshape(_NW, nch, _K)
  dst3 = edge_index[1].reshape(_NW, nch, _K)
  xp = jnp.concatenate(
      [x, jnp.zeros((npd - n, x.shape[1]), jnp.float32)], axis=0)

  wa, wb, c = _fold(params["enc0"])
  a, b = _tc_pre0(xp, wa, wb, c)
  s, degp = _edge_pass(a, b, dst3, src3, with_deg=True)

  wa, wb, c = _fold(params["enc1"])
  a, b, invd = _tc_postpre_deg(s, degp.T, wa, wb, c)
  s = _edge_pass(a, b, dst3, src3, with_deg=False)

  wa, wb, c = _fold(params["enc2"])
  a, b = _tc_postpre(s, invd, wa, wb, c)
  s = _edge_pass(a, b, dst3, src3, with_deg=False)

  bp = jnp.concatenate(
      [batch_index, jnp.full((npd - n,), g, jnp.int32)], axis=0)
  pool = _pool_pass(s, invd[:, 0].reshape(_NW, 1, 320),
                    bp.reshape(_NW, 1, 320), g4)

  fc1, fc2 = params["fc1"], params["fc2"]
  dfc1, dfc2 = params["dfc1"], params["dfc2"]
  wa0, wb0, c0 = _fold(params["dec0"])
  nf = fc1["W"].shape[0] // 2
  z4, a0s, b0s = _tc_mid(
      pool, fc1["W"][:nf], fc1["W"][nf:], fc1["b"].reshape(1, -1),
      fc2["W"], fc2["b"].reshape(1, -1),
      dfc1["W"], dfc1["b"].reshape(1, -1),
      dfc2["W"], dfc2["b"].reshape(1, -1),
      wa0, wb0, c0)
  z = z4[:g]

  a = jnp.repeat(a0s[:g], n // g, axis=0)
  b = jnp.repeat(b0s[:g], n // g, axis=0)
  a = jnp.concatenate(
      [a, jnp.zeros((npd - n, a.shape[1]), jnp.float32)], axis=0)
  b = jnp.concatenate(
      [b, jnp.zeros((npd - n, b.shape[1]), jnp.float32)], axis=0)
  s = _edge_pass(a, b, dst3, src3, with_deg=False)

  wa, wb, c = _fold(params["dec1"])
  a, b = _tc_postpre(s, invd, wa, wb, c)
  s = _edge_pass(a, b, dst3, src3, with_deg=False)

  wa, wb, c = _fold(params["dec2"])
  a, b = _tc_postpre(s, invd, wa, wb, c)
  sl, sr = _edge_pass(a[:, :64], b[:, :64], dst3, src3, with_deg=False,
                      ab2=(a[:, 64:], b[:, 64:]))
  xd = _tc_post2(sl, sr, invd)[:n]
  return xd, z


# speedup vs baseline: 1.0049x; 1.0049x over previous
import functools

import jax
import jax.numpy as jnp
import numpy as np
from jax import lax
from jax.experimental import pallas as pl
from jax.experimental.pallas import tpu as pltpu
from jax.experimental.pallas import tpu_sc as plsc

_BN_EPS = 1e-5
_NC = 2
_NS = 16
_NW = _NC * _NS
_K = 80


def _fold(p):
  W, b, g, be = p["W"], p["b"], p["g"], p["be"]
  ci = W.shape[0] // 2
  s = (g / np.sqrt(1.0 + _BN_EPS)).astype(jnp.float32)
  wa = (W[:ci] - W[ci:]) * s[None, :]
  wb = W[ci:] * s[None, :]
  c = b * s + be
  return wa, wb, c.reshape(1, -1)


def _edge_pass(a, b, dst3, src3, with_deg, ab2=None):
  n_nodes, co = a.shape
  nch = dst3.shape[1]
  npt = n_nodes // _NS
  nzc = 5
  nzr = npt // nzc
  cg = co // 16
  mesh = plsc.VectorSubcoreMesh(core_axis_name="c", subcore_axis_name="s")

  out_type = [jax.ShapeDtypeStruct((_NC, n_nodes, co), jnp.float32)]
  if ab2 is not None:
    out_type.append(jax.ShapeDtypeStruct((_NC, n_nodes, co), jnp.float32))
  scratch = [
      pltpu.VMEM((nch, _K), jnp.int32),
      pltpu.VMEM((nch, _K), jnp.int32),
      pltpu.VMEM((_K, co), jnp.float32),
      pltpu.VMEM((_K, co), jnp.float32),
      pltpu.VMEM((_K, co), jnp.float32),
      pltpu.VMEM((_K, co), jnp.float32),
      pltpu.VMEM((_K, co), jnp.float32),
      pltpu.VMEM((_K, co), jnp.float32),
      pltpu.VMEM((nzr, co), jnp.float32),
      pltpu.VMEM_SHARED((n_nodes, co), jnp.float32),
      pltpu.SemaphoreType.DMA,
      pltpu.SemaphoreType.DMA,
      pltpu.SemaphoreType.DMA,
      pltpu.SemaphoreType.DMA,
  ]
  if with_deg:
    out_type.append(jax.ShapeDtypeStruct((_NW, n_nodes), jnp.float32))
    scratch.append(pltpu.VMEM((n_nodes,), jnp.float32))

  @functools.partial(
      pl.kernel,
      out_type=tuple(out_type) if len(out_type) > 1 else out_type[0],
      mesh=mesh,
      compiler_params=pltpu.CompilerParams(
          use_tc_tiling_on_sc=False, needs_layout_passes=False),
      scratch_types=scratch,
  )
  def ek(*refs):
    n_in = 4 if ab2 is None else 6
    n_out = len(out_type)
    ins, outs, rest = refs[:n_in], refs[n_in:n_in + n_out], refs[n_in + n_out:]
    dst_hbm, src_hbm = ins[n_in - 2], ins[n_in - 1]
    if with_deg:
      (dstv, srcv, bufa0, bufb0, bufa1, bufb1, bufm0, bufm1,
       zbuf, acc, sem0, sem1, ssc0, ssc1, degacc) = rest
    else:
      (dstv, srcv, bufa0, bufb0, bufa1, bufb1, bufm0, bufm1,
       zbuf, acc, sem0, sem1, ssc0, ssc1) = rest
    cid = lax.axis_index("c")
    sid = lax.axis_index("s")
    wid = sid * _NC + cid
    row0 = sid * npt
    pltpu.sync_copy(dst_hbm.at[wid], dstv)
    pltpu.sync_copy(src_hbm.at[wid], srcv)

    z16 = jnp.zeros((16,), jnp.float32)

    def zero_acc():
      def zrow(i, _):
        def zcol(j, _):
          zbuf[i, pl.ds(j * 16, 16)] = z16
          return 0
        return lax.fori_loop(0, cg, zcol, 0)
      lax.fori_loop(0, nzr, zrow, 0)
      for q in range(nzc):
        pltpu.sync_copy(zbuf, acc.at[pl.ds(row0 + q * nzr, nzr)])

    zero_acc()

    if with_deg:
      deg_hbm = outs[1]

      def zdeg(i, _):
        degacc[pl.ds(i * 16, 16)] = z16
        return 0
      lax.fori_loop(0, n_nodes // 16, zdeg, 0)
      one16 = jnp.ones((16,), jnp.float32)

      def drow(kc, _):
        for gq in range(_K // 16):
          idx = dstv[kc, pl.ds(gq * 16, 16)]
          plsc.addupdate_scatter(degacc, [idx], one16)
        return 0
      lax.fori_loop(0, nch, drow, 0)
      pltpu.sync_copy(degacc, deg_hbm.at[wid])

    def run_pass(a_hbm, b_hbm, out_hbm):
      plsc.subcore_barrier()

      def issue(kc, ba, bb, sem):
        pltpu.async_copy(a_hbm.at[dstv.at[kc]], ba, sem)
        pltpu.async_copy(b_hbm.at[srcv.at[kc]], bb, sem)

      def wait(ba, bb, sem):
        pltpu.make_async_copy(a_hbm.at[dstv.at[0]], ba, sem).wait()
        pltpu.make_async_copy(b_hbm.at[srcv.at[0]], bb, sem).wait()

      def compute(ba, bb, bm):
        def erow(t, _):
          for u in range(4):
            i = t * 4 + u
            for j in range(cg):
              d = pl.ds(j * 16, 16)
              bm[i, d] = jnp.maximum(ba[i, d] + bb[i, d], 0.0)
          return 0
        lax.fori_loop(0, _K // 4, erow, 0)

      def iscatter(kc, bm, sem):
        pltpu.async_copy(bm, acc.at[dstv.at[kc]], sem, add=True)

      def wscatter(bm, sem):
        pltpu.make_async_copy(bm, acc.at[dstv.at[0]], sem).wait()

      issue(0, bufa0, bufb0, sem0)
      issue(1, bufa1, bufb1, sem1)
      wait(bufa0, bufb0, sem0)
      compute(bufa0, bufb0, bufm0)
      iscatter(0, bufm0, ssc0)
      issue(2, bufa0, bufb0, sem0)
      wait(bufa1, bufb1, sem1)
      compute(bufa1, bufb1, bufm1)
      iscatter(1, bufm1, ssc1)

      def pair(p, _):
        kc0 = 2 * p
        issue(kc0 + 1, bufa1, bufb1, sem1)
        wait(bufa0, bufb0, sem0)
        wscatter(bufm0, ssc0)
        compute(bufa0, bufb0, bufm0)
        iscatter(kc0, bufm0, ssc0)
        issue(kc0 + 2, bufa0, bufb0, sem0)
        wait(bufa1, bufb1, sem1)
        wscatter(bufm1, ssc1)
        compute(bufa1, bufb1, bufm1)
        iscatter(kc0 + 1, bufm1, ssc1)
        return 0
      lax.fori_loop(1, (nch - 1) // 2, pair, 0)
      wait(bufa0, bufb0, sem0)
      wscatter(bufm0, ssc0)
      compute(bufa0, bufb0, bufm0)
      iscatter(nch - 1, bufm0, ssc0)
      wscatter(bufm0, ssc0)
      wscatter(bufm1, ssc1)

      plsc.subcore_barrier()
      pltpu.sync_copy(acc.at[pl.ds(row0, npt)],
                      out_hbm.at[cid, pl.ds(row0, npt)])

    run_pass(ins[0], ins[1], outs[0])
    if ab2 is not None:
      zero_acc()
      run_pass(ins[2], ins[3], outs[1])

  if ab2 is None:
    return ek(a, b, dst3, src3)
  return ek(a, b, ab2[0], ab2[1], dst3, src3)


def _pool_pass(s3, inv3, bp3, g4):
  npn = bp3.shape[2]
  mesh = plsc.VectorSubcoreMesh(core_axis_name="c", subcore_axis_name="s")

  @functools.partial(
      pl.kernel,
      out_type=jax.ShapeDtypeStruct((_NW, 3, g4, 16), jnp.float32),
      mesh=mesh,
      compiler_params=pltpu.CompilerParams(
          use_tc_tiling_on_sc=False, needs_layout_passes=False),
      scratch_types=[
          pltpu.VMEM((npn, 16), jnp.float32),
          pltpu.VMEM((npn, 16), jnp.float32),
          pltpu.VMEM((1, npn), jnp.float32),
          pltpu.VMEM((1, npn), jnp.int32),
          pltpu.VMEM((g4, 16), jnp.float32),
          pltpu.VMEM((g4, 16), jnp.float32),
          pltpu.VMEM((g4, 16), jnp.float32),
      ],
  )
  def pk(s_hbm, inv_hbm, b_hbm, out_hbm, s0buf, s1buf, ibuf, bbuf,
         sacc, cacc, macc):
    cid = lax.axis_index("c")
    sid = lax.axis_index("s")
    wid = sid * _NC + cid
    w0 = wid * npn
    pltpu.sync_copy(s_hbm.at[0, pl.ds(w0, npn)], s0buf)
    pltpu.sync_copy(s_hbm.at[1, pl.ds(w0, npn)], s1buf)
    pltpu.sync_copy(inv_hbm.at[wid], ibuf)
    pltpu.sync_copy(b_hbm.at[wid], bbuf)

    z16 = jnp.zeros((16,), jnp.float32)
    ninf = jnp.full((16,), -jnp.inf, jnp.float32)

    def init(i, _):
      sacc[i, :] = z16
      cacc[i, :] = z16
      macc[i, :] = ninf
      return 0
    lax.fori_loop(0, g4, init, 0)

    col = lax.iota(jnp.int32, 16)
    one16 = jnp.ones((16,), jnp.float32)

    def node16(t, _):
      bidx = bbuf[0, pl.ds(t * 16, 16)]
      invs = ibuf[0, pl.ds(t * 16, 16)]
      for lane in range(16):
        bb = jnp.full((16,), bidx[lane], jnp.int32)
        i = t * 16 + lane
        hrow = jnp.maximum((s0buf[i, :] + s1buf[i, :]) * invs[lane], 0.0)
        s = plsc.load_gather(sacc, [bb, col])
        plsc.store_scatter(sacc, [bb, col], s + hrow)
        c = plsc.load_gather(cacc, [bb, col])
        plsc.store_scatter(cacc, [bb, col], c + one16)
        m = plsc.load_gather(macc, [bb, col])
        plsc.store_scatter(macc, [bb, col], jnp.maximum(m, hrow))
      return 0
    lax.fori_loop(0, npn // 16, node16, 0)

    pltpu.sync_copy(sacc, out_hbm.at[wid, 0])
    pltpu.sync_copy(cacc, out_hbm.at[wid, 1])
    pltpu.sync_copy(macc, out_hbm.at[wid, 2])

  return pk(s3, inv3, bp3)


_BLK = 2560


def _tc_pre0(x, wa, wb, c):
  n, ci = x.shape
  co = wa.shape[1]

  def body(x_ref, wa_ref, wb_ref, c_ref, a_ref, b_ref):
    h = x_ref[...]
    a_ref[...] = jnp.dot(h, wa_ref[...],
                         preferred_element_type=jnp.float32) + c_ref[...]
    b_ref[...] = jnp.dot(h, wb_ref[...], preferred_element_type=jnp.float32)

  return pl.pallas_call(
      body,
      grid=(n // _BLK,),
      in_specs=[
          pl.BlockSpec((_BLK, ci), lambda i: (i, 0)),
          pl.BlockSpec((ci, co), lambda i: (0, 0)),
          pl.BlockSpec((ci, co), lambda i: (0, 0)),
          pl.BlockSpec((1, co), lambda i: (0, 0)),
      ],
      out_specs=[pl.BlockSpec((_BLK, co), lambda i: (i, 0)),
                 pl.BlockSpec((_BLK, co), lambda i: (i, 0))],
      out_shape=[jax.ShapeDtypeStruct((n, co), jnp.float32)] * 2,
  )(x, wa, wb, c)


def _tc_postpre_deg(s, degp, wa, wb, c):
  n = s.shape[1]
  cp, co = wa.shape

  def body(s_ref, deg_ref, wa_ref, wb_ref, c_ref, a_ref, b_ref, inv_ref):
    st = s_ref[0] + s_ref[1]
    deg = jnp.sum(deg_ref[...], axis=1, keepdims=True)
    invd = 1.0 / jnp.maximum(deg, 1.0)
    h = jnp.maximum(st * invd, 0.0)
    a_ref[...] = jnp.dot(h, wa_ref[...],
                         preferred_element_type=jnp.float32) + c_ref[...]
    b_ref[...] = jnp.dot(h, wb_ref[...], preferred_element_type=jnp.float32)
    inv_ref[...] = invd

  return pl.pallas_call(
      body,
      grid=(n // _BLK,),
      in_specs=[
          pl.BlockSpec((2, _BLK, cp), lambda i: (0, i, 0)),
          pl.BlockSpec((_BLK, _NW), lambda i: (i, 0)),
          pl.BlockSpec((cp, co), lambda i: (0, 0)),
          pl.BlockSpec((cp, co), lambda i: (0, 0)),
          pl.BlockSpec((1, co), lambda i: (0, 0)),
      ],
      out_specs=[pl.BlockSpec((_BLK, co), lambda i: (i, 0)),
                 pl.BlockSpec((_BLK, co), lambda i: (i, 0)),
                 pl.BlockSpec((_BLK, 1), lambda i: (i, 0))],
      out_shape=[jax.ShapeDtypeStruct((n, co), jnp.float32),
                 jax.ShapeDtypeStruct((n, co), jnp.float32),
                 jax.ShapeDtypeStruct((n, 1), jnp.float32)],
  )(s, degp, wa, wb, c)


def _tc_postpre(s, invd, wa, wb, c):
  n, cp = s.shape[1], s.shape[2]
  co = wa.shape[1]

  def body(s_ref, inv_ref, wa_ref, wb_ref, c_ref, a_ref, b_ref):
    h = jnp.maximum((s_ref[0] + s_ref[1]) * inv_ref[...], 0.0)
    a_ref[...] = jnp.dot(h, wa_ref[...],
                         preferred_element_type=jnp.float32) + c_ref[...]
    b_ref[...] = jnp.dot(h, wb_ref[...], preferred_element_type=jnp.float32)

  return pl.pallas_call(
      body,
      grid=(n // _BLK,),
      in_specs=[
          pl.BlockSpec((2, _BLK, cp), lambda i: (0, i, 0)),
          pl.BlockSpec((_BLK, 1), lambda i: (i, 0)),
          pl.BlockSpec((cp, co), lambda i: (0, 0)),
          pl.BlockSpec((cp, co), lambda i: (0, 0)),
          pl.BlockSpec((1, co), lambda i: (0, 0)),
      ],
      out_specs=[pl.BlockSpec((_BLK, co), lambda i: (i, 0)),
                 pl.BlockSpec((_BLK, co), lambda i: (i, 0))],
      out_shape=[jax.ShapeDtypeStruct((n, co), jnp.float32)] * 2,
  )(s, invd, wa, wb, c)


def _tc_post2(sl, sr, invd):
  n, cp = sl.shape[1], sl.shape[2]

  def body(sl_ref, sr_ref, inv_ref, o_ref):
    hl = jnp.maximum((sl_ref[0] + sl_ref[1]) * inv_ref[...], 0.0)
    hr = jnp.maximum((sr_ref[0] + sr_ref[1]) * inv_ref[...], 0.0)
    o_ref[...] = jnp.concatenate([hl, hr], axis=1)

  return pl.pallas_call(
      body,
      grid=(n // _BLK,),
      in_specs=[
          pl.BlockSpec((2, _BLK, cp), lambda i: (0, i, 0)),
          pl.BlockSpec((2, _BLK, cp), lambda i: (0, i, 0)),
          pl.BlockSpec((_BLK, 1), lambda i: (i, 0)),
      ],
      out_specs=pl.BlockSpec((_BLK, 2 * cp), lambda i: (i, 0)),
      out_shape=jax.ShapeDtypeStruct((n, 2 * cp), jnp.float32),
  )(sl, sr, invd)


def _tc_post(s, invd):
  n, cp = s.shape[1], s.shape[2]

  def body(s_ref, inv_ref, o_ref):
    o_ref[...] = jnp.maximum((s_ref[0] + s_ref[1]) * inv_ref[...], 0.0)

  return pl.pallas_call(
      body,
      grid=(n // _BLK,),
      in_specs=[
          pl.BlockSpec((2, _BLK, cp), lambda i: (0, i, 0)),
          pl.BlockSpec((_BLK, 1), lambda i: (i, 0)),
      ],
      out_specs=pl.BlockSpec((_BLK, cp), lambda i: (i, 0)),
      out_shape=jax.ShapeDtypeStruct((n, cp), jnp.float32),
  )(s, invd)


def _tc_mid(pool, w1a, w1b, b1, w2, b2, wd1, bd1, wd2, bd2, wa0, wb0, c0):
  g4 = pool.shape[2]
  co0 = wa0.shape[1]

  def body(p_ref, w1a_r, w1b_r, b1_r, w2_r, b2_r, wd1_r, bd1_r, wd2_r,
           bd2_r, wa0_r, wb0_r, c0_r, z_ref, a0_ref, b0_ref):
    def step(t, carry):
      su, cn, mx = carry
      return (su + p_ref[t, 0], cn + p_ref[t, 1],
              jnp.maximum(mx, p_ref[t, 2]))
    su, cn, mx = lax.fori_loop(
        0, _NW, step,
        (jnp.zeros((g4, 16), jnp.float32), jnp.zeros((g4, 16), jnp.float32),
         jnp.full((g4, 16), -jnp.inf, jnp.float32)))
    xm = su / jnp.maximum(cn, 1.0)
    g = jnp.maximum(
        jnp.dot(xm, w1a_r[...], preferred_element_type=jnp.float32)
        + jnp.dot(mx, w1b_r[...], preferred_element_type=jnp.float32)
        + b1_r[...], 0.0)
    z = jnp.maximum(
        jnp.dot(g, w2_r[...], preferred_element_type=jnp.float32)
        + b2_r[...], 0.0)
    z_ref[...] = z
    y = jnp.maximum(
        jnp.dot(z, wd1_r[...], preferred_element_type=jnp.float32)
        + bd1_r[...], 0.0)
    y = jnp.maximum(
        jnp.dot(y, wd2_r[...], preferred_element_type=jnp.float32)
        + bd2_r[...], 0.0)
    a0_ref[...] = jnp.dot(y, wa0_r[...],
                          preferred_element_type=jnp.float32) + c0_r[...]
    b0_ref[...] = jnp.dot(y, wb0_r[...], preferred_element_type=jnp.float32)

  return pl.pallas_call(
      body,
      out_shape=[jax.ShapeDtypeStruct((g4, 16), jnp.float32),
                 jax.ShapeDtypeStruct((g4, co0), jnp.float32),
                 jax.ShapeDtypeStruct((g4, co0), jnp.float32)],
  )(pool, w1a, w1b, b1, w2, b2, wd1, bd1, wd2, bd2, wa0, wb0, c0)


def kernel(x, params, edge_index, batch_index):
  n = x.shape[0]
  e = edge_index.shape[1]
  g = 100
  g4 = 104
  npd = _NW * 320
  nch = e // (_NW * _K)
  src3 = edge_index[0].reshape(_NW, nch, _K)
  dst3 = edge_index[1].reshape(_NW, nch, _K)
  xp = jnp.concatenate(
      [x, jnp.zeros((npd - n, x.shape[1]), jnp.float32)], axis=0)

  wa, wb, c = _fold(params["enc0"])
  a, b = _tc_pre0(xp, wa, wb, c)
  s, degp = _edge_pass(a, b, dst3, src3, with_deg=True)

  wa, wb, c = _fold(params["enc1"])
  a, b, invd = _tc_postpre_deg(s, degp.T, wa, wb, c)
  s = _edge_pass(a, b, dst3, src3, with_deg=False)

  wa, wb, c = _fold(params["enc2"])
  a, b = _tc_postpre(s, invd, wa, wb, c)
  s = _edge_pass(a, b, dst3, src3, with_deg=False)

  bp = jnp.concatenate(
      [batch_index, jnp.full((npd - n,), g, jnp.int32)], axis=0)
  pool = _pool_pass(s, invd[:, 0].reshape(_NW, 1, 320),
                    bp.reshape(_NW, 1, 320), g4)

  fc1, fc2 = params["fc1"], params["fc2"]
  dfc1, dfc2 = params["dfc1"], params["dfc2"]
  wa0, wb0, c0 = _fold(params["dec0"])
  nf = fc1["W"].shape[0] // 2
  z4, a0s, b0s = _tc_mid(
      pool, fc1["W"][:nf], fc1["W"][nf:], fc1["b"].reshape(1, -1),
      fc2["W"], fc2["b"].reshape(1, -1),
      dfc1["W"], dfc1["b"].reshape(1, -1),
      dfc2["W"], dfc2["b"].reshape(1, -1),
      wa0, wb0, c0)
  z = z4[:g]

  a = jnp.repeat(a0s[:g], n // g, axis=0)
  b = jnp.repeat(b0s[:g], n // g, axis=0)
  a = jnp.concatenate(
      [a, jnp.zeros((npd - n, a.shape[1]), jnp.float32)], axis=0)
  b = jnp.concatenate(
      [b, jnp.zeros((npd - n, b.shape[1]), jnp.float32)], axis=0)
  s = _edge_pass(a, b, dst3, src3, with_deg=False)

  wa, wb, c = _fold(params["dec1"])
  a, b = _tc_postpre(s, invd, wa, wb, c)
  s = _edge_pass(a, b, dst3, src3, with_deg=False)

  wa, wb, c = _fold(params["dec2"])
  a, b = _tc_postpre(s, invd, wa, wb, c)
  sl, sr = _edge_pass(a[:, :64], b[:, :64], dst3, src3, with_deg=False,
                      ab2=(a[:, 64:], b[:, 64:]))
  xd = _tc_post2(sl, sr, invd)[:n]
  return xd, z


# speedup vs baseline: 1.0057x; 1.0008x over previous
import functools

import jax
import jax.numpy as jnp
import numpy as np
from jax import lax
from jax.experimental import pallas as pl
from jax.experimental.pallas import tpu as pltpu
from jax.experimental.pallas import tpu_sc as plsc

_BN_EPS = 1e-5
_NC = 2
_NS = 16
_NW = _NC * _NS
_K = 80


def _fold(p):
  W, b, g, be = p["W"], p["b"], p["g"], p["be"]
  ci = W.shape[0] // 2
  s = (g / np.sqrt(1.0 + _BN_EPS)).astype(jnp.float32)
  wa = (W[:ci] - W[ci:]) * s[None, :]
  wb = W[ci:] * s[None, :]
  c = b * s + be
  return wa, wb, c.reshape(1, -1)


def _edge_pass(a, b, dst3, src3, with_deg, ab2=None):
  n_nodes, co = a.shape
  nch = dst3.shape[1]
  npt = n_nodes // _NS
  nzc = 5
  nzr = npt // nzc
  cg = co // 16
  mesh = plsc.VectorSubcoreMesh(core_axis_name="c", subcore_axis_name="s")

  out_type = [jax.ShapeDtypeStruct((_NC, n_nodes, co), jnp.float32)]
  if ab2 is not None:
    out_type.append(jax.ShapeDtypeStruct((_NC, n_nodes, co), jnp.float32))
  scratch = [
      pltpu.VMEM((nch, _K), jnp.int32),
      pltpu.VMEM((nch, _K), jnp.int32),
      pltpu.VMEM((_K, co), jnp.float32),
      pltpu.VMEM((_K, co), jnp.float32),
      pltpu.VMEM((_K, co), jnp.float32),
      pltpu.VMEM((_K, co), jnp.float32),
      pltpu.VMEM((_K, co), jnp.float32),
      pltpu.VMEM((_K, co), jnp.float32),
      pltpu.VMEM((nzr, co), jnp.float32),
      pltpu.VMEM_SHARED((n_nodes, co), jnp.float32),
      pltpu.SemaphoreType.DMA,
      pltpu.SemaphoreType.DMA,
      pltpu.SemaphoreType.DMA,
      pltpu.SemaphoreType.DMA,
  ]
  if with_deg:
    out_type.append(jax.ShapeDtypeStruct((_NW, n_nodes), jnp.float32))
    scratch.append(pltpu.VMEM((n_nodes,), jnp.float32))

  @functools.partial(
      pl.kernel,
      out_type=tuple(out_type) if len(out_type) > 1 else out_type[0],
      mesh=mesh,
      compiler_params=pltpu.CompilerParams(
          use_tc_tiling_on_sc=False, needs_layout_passes=False),
      scratch_types=scratch,
  )
  def ek(*refs):
    n_in = 4 if ab2 is None else 6
    n_out = len(out_type)
    ins, outs, rest = refs[:n_in], refs[n_in:n_in + n_out], refs[n_in + n_out:]
    dst_hbm, src_hbm = ins[n_in - 2], ins[n_in - 1]
    if with_deg:
      (dstv, srcv, bufa0, bufb0, bufa1, bufb1, bufm0, bufm1,
       zbuf, acc, sem0, sem1, ssc0, ssc1, degacc) = rest
    else:
      (dstv, srcv, bufa0, bufb0, bufa1, bufb1, bufm0, bufm1,
       zbuf, acc, sem0, sem1, ssc0, ssc1) = rest
    cid = lax.axis_index("c")
    sid = lax.axis_index("s")
    wid = sid * _NC + cid
    row0 = sid * npt
    pltpu.sync_copy(dst_hbm.at[wid], dstv)
    pltpu.sync_copy(src_hbm.at[wid], srcv)

    z16 = jnp.zeros((16,), jnp.float32)

    def zero_acc():
      def zrow(i, _):
        def zcol(j, _):
          zbuf[i, pl.ds(j * 16, 16)] = z16
          return 0
        return lax.fori_loop(0, cg, zcol, 0)
      lax.fori_loop(0, nzr, zrow, 0)
      for q in range(nzc):
        pltpu.sync_copy(zbuf, acc.at[pl.ds(row0 + q * nzr, nzr)])

    zero_acc()

    if with_deg:
      deg_hbm = outs[1]

      def zdeg(i, _):
        degacc[pl.ds(i * 16, 16)] = z16
        return 0
      lax.fori_loop(0, n_nodes // 16, zdeg, 0)
      one16 = jnp.ones((16,), jnp.float32)

      def drow(kc, _):
        for gq in range(_K // 16):
          idx = dstv[kc, pl.ds(gq * 16, 16)]
          plsc.addupdate_scatter(degacc, [idx], one16)
        return 0
      lax.fori_loop(0, nch, drow, 0)
      pltpu.sync_copy(degacc, deg_hbm.at[wid])

    def run_pass(a_hbm, b_hbm, out_hbm):
      plsc.subcore_barrier()

      def issue(kc, ba, bb, sem):
        pltpu.async_copy(a_hbm.at[dstv.at[kc]], ba, sem)
        pltpu.async_copy(b_hbm.at[srcv.at[kc]], bb, sem)

      def wait(ba, bb, sem):
        pltpu.make_async_copy(a_hbm.at[dstv.at[0]], ba, sem).wait()
        pltpu.make_async_copy(b_hbm.at[srcv.at[0]], bb, sem).wait()

      def compute(ba, bb, bm):
        def erow(t, _):
          for u in range(4):
            i = t * 4 + u
            for j in range(cg):
              d = pl.ds(j * 16, 16)
              bm[i, d] = jnp.maximum(ba[i, d] + bb[i, d], 0.0)
          return 0
        lax.fori_loop(0, _K // 4, erow, 0)

      def iscatter(kc, bm, sem):
        pltpu.async_copy(bm, acc.at[dstv.at[kc]], sem, add=True)

      def wscatter(bm, sem):
        pltpu.make_async_copy(bm, acc.at[dstv.at[0]], sem).wait()

      issue(0, bufa0, bufb0, sem0)
      issue(1, bufa1, bufb1, sem1)
      wait(bufa0, bufb0, sem0)
      compute(bufa0, bufb0, bufm0)
      iscatter(0, bufm0, ssc0)
      issue(2, bufa0, bufb0, sem0)
      wait(bufa1, bufb1, sem1)
      compute(bufa1, bufb1, bufm1)
      iscatter(1, bufm1, ssc1)

      def pair(p, _):
        kc0 = 2 * p
        issue(kc0 + 1, bufa1, bufb1, sem1)
        wait(bufa0, bufb0, sem0)
        wscatter(bufm0, ssc0)
        compute(bufa0, bufb0, bufm0)
        iscatter(kc0, bufm0, ssc0)
        issue(kc0 + 2, bufa0, bufb0, sem0)
        wait(bufa1, bufb1, sem1)
        wscatter(bufm1, ssc1)
        compute(bufa1, bufb1, bufm1)
        iscatter(kc0 + 1, bufm1, ssc1)
        return 0
      lax.fori_loop(1, (nch - 1) // 2, pair, 0)
      wait(bufa0, bufb0, sem0)
      wscatter(bufm0, ssc0)
      compute(bufa0, bufb0, bufm0)
      iscatter(nch - 1, bufm0, ssc0)
      wscatter(bufm0, ssc0)
      wscatter(bufm1, ssc1)

      plsc.subcore_barrier()
      pltpu.sync_copy(acc.at[pl.ds(row0, npt)],
                      out_hbm.at[cid, pl.ds(row0, npt)])

    run_pass(ins[0], ins[1], outs[0])
    if ab2 is not None:
      zero_acc()
      run_pass(ins[2], ins[3], outs[1])

  if ab2 is None:
    return ek(a, b, dst3, src3)
  return ek(a, b, ab2[0], ab2[1], dst3, src3)


def _pool_pass(s3, inv3, bp3, g4):
  npn = bp3.shape[2]
  mesh = plsc.VectorSubcoreMesh(core_axis_name="c", subcore_axis_name="s")

  @functools.partial(
      pl.kernel,
      out_type=jax.ShapeDtypeStruct((_NW, 3, g4, 16), jnp.float32),
      mesh=mesh,
      compiler_params=pltpu.CompilerParams(
          use_tc_tiling_on_sc=False, needs_layout_passes=False),
      scratch_types=[
          pltpu.VMEM((npn, 16), jnp.float32),
          pltpu.VMEM((npn, 16), jnp.float32),
          pltpu.VMEM((1, npn), jnp.float32),
          pltpu.VMEM((1, npn), jnp.int32),
          pltpu.VMEM((g4, 16), jnp.float32),
          pltpu.VMEM((g4, 16), jnp.float32),
          pltpu.VMEM((g4, 16), jnp.float32),
      ],
  )
  def pk(s_hbm, inv_hbm, b_hbm, out_hbm, s0buf, s1buf, ibuf, bbuf,
         sacc, cacc, macc):
    cid = lax.axis_index("c")
    sid = lax.axis_index("s")
    wid = sid * _NC + cid
    w0 = wid * npn
    pltpu.sync_copy(s_hbm.at[0, pl.ds(w0, npn)], s0buf)
    pltpu.sync_copy(s_hbm.at[1, pl.ds(w0, npn)], s1buf)
    pltpu.sync_copy(inv_hbm.at[wid], ibuf)
    pltpu.sync_copy(b_hbm.at[wid], bbuf)

    z16 = jnp.zeros((16,), jnp.float32)
    ninf = jnp.full((16,), -jnp.inf, jnp.float32)

    def init(i, _):
      sacc[i, :] = z16
      cacc[i, :] = z16
      macc[i, :] = ninf
      return 0
    lax.fori_loop(0, g4, init, 0)

    col = lax.iota(jnp.int32, 16)
    one16 = jnp.ones((16,), jnp.float32)

    def node16(t, _):
      bidx = bbuf[0, pl.ds(t * 16, 16)]
      invs = ibuf[0, pl.ds(t * 16, 16)]
      for lane in range(16):
        bb = jnp.full((16,), bidx[lane], jnp.int32)
        i = t * 16 + lane
        hrow = jnp.maximum((s0buf[i, :] + s1buf[i, :]) * invs[lane], 0.0)
        s = plsc.load_gather(sacc, [bb, col])
        plsc.store_scatter(sacc, [bb, col], s + hrow)
        c = plsc.load_gather(cacc, [bb, col])
        plsc.store_scatter(cacc, [bb, col], c + one16)
        m = plsc.load_gather(macc, [bb, col])
        plsc.store_scatter(macc, [bb, col], jnp.maximum(m, hrow))
      return 0
    lax.fori_loop(0, npn // 16, node16, 0)

    pltpu.sync_copy(sacc, out_hbm.at[wid, 0])
    pltpu.sync_copy(cacc, out_hbm.at[wid, 1])
    pltpu.sync_copy(macc, out_hbm.at[wid, 2])

  return pk(s3, inv3, bp3)


_BLK = 2560


def _tc_pre0(x, wa, wb, c):
  n, ci = x.shape
  co = wa.shape[1]

  def body(x_ref, wa_ref, wb_ref, c_ref, a_ref, b_ref):
    h = x_ref[...]
    a_ref[...] = jnp.dot(h, wa_ref[...],
                         preferred_element_type=jnp.float32) + c_ref[...]
    b_ref[...] = jnp.dot(h, wb_ref[...], preferred_element_type=jnp.float32)

  return pl.pallas_call(
      body,
      grid=(n // _BLK,),
      in_specs=[
          pl.BlockSpec((_BLK, ci), lambda i: (i, 0)),
          pl.BlockSpec((ci, co), lambda i: (0, 0)),
          pl.BlockSpec((ci, co), lambda i: (0, 0)),
          pl.BlockSpec((1, co), lambda i: (0, 0)),
      ],
      out_specs=[pl.BlockSpec((_BLK, co), lambda i: (i, 0)),
                 pl.BlockSpec((_BLK, co), lambda i: (i, 0))],
      out_shape=[jax.ShapeDtypeStruct((n, co), jnp.float32)] * 2,
  )(x, wa, wb, c)


def _tc_postpre_deg(s, degp, wa, wb, c):
  n = s.shape[1]
  cp, co = wa.shape

  def body(s_ref, deg_ref, wa_ref, wb_ref, c_ref, a_ref, b_ref, inv_ref):
    st = s_ref[0] + s_ref[1]
    deg = jnp.sum(deg_ref[...], axis=1, keepdims=True)
    invd = 1.0 / jnp.maximum(deg, 1.0)
    h = jnp.maximum(st * invd, 0.0)
    a_ref[...] = jnp.dot(h, wa_ref[...],
                         preferred_element_type=jnp.float32) + c_ref[...]
    b_ref[...] = jnp.dot(h, wb_ref[...], preferred_element_type=jnp.float32)
    inv_ref[...] = invd

  return pl.pallas_call(
      body,
      grid=(n // _BLK,),
      in_specs=[
          pl.BlockSpec((2, _BLK, cp), lambda i: (0, i, 0)),
          pl.BlockSpec((_BLK, _NW), lambda i: (i, 0)),
          pl.BlockSpec((cp, co), lambda i: (0, 0)),
          pl.BlockSpec((cp, co), lambda i: (0, 0)),
          pl.BlockSpec((1, co), lambda i: (0, 0)),
      ],
      out_specs=[pl.BlockSpec((_BLK, co), lambda i: (i, 0)),
                 pl.BlockSpec((_BLK, co), lambda i: (i, 0)),
                 pl.BlockSpec((_BLK, 1), lambda i: (i, 0))],
      out_shape=[jax.ShapeDtypeStruct((n, co), jnp.float32),
                 jax.ShapeDtypeStruct((n, co), jnp.float32),
                 jax.ShapeDtypeStruct((n, 1), jnp.float32)],
  )(s, degp, wa, wb, c)


def _tc_postpre(s, invd, wa, wb, c):
  n, cp = s.shape[1], s.shape[2]
  co = wa.shape[1]

  def body(s_ref, inv_ref, wa_ref, wb_ref, c_ref, a_ref, b_ref):
    h = jnp.maximum((s_ref[0] + s_ref[1]) * inv_ref[...], 0.0)
    a_ref[...] = jnp.dot(h, wa_ref[...],
                         preferred_element_type=jnp.float32) + c_ref[...]
    b_ref[...] = jnp.dot(h, wb_ref[...], preferred_element_type=jnp.float32)

  return pl.pallas_call(
      body,
      grid=(n // _BLK,),
      in_specs=[
          pl.BlockSpec((2, _BLK, cp), lambda i: (0, i, 0)),
          pl.BlockSpec((_BLK, 1), lambda i: (i, 0)),
          pl.BlockSpec((cp, co), lambda i: (0, 0)),
          pl.BlockSpec((cp, co), lambda i: (0, 0)),
          pl.BlockSpec((1, co), lambda i: (0, 0)),
      ],
      out_specs=[pl.BlockSpec((_BLK, co), lambda i: (i, 0)),
                 pl.BlockSpec((_BLK, co), lambda i: (i, 0))],
      out_shape=[jax.ShapeDtypeStruct((n, co), jnp.float32)] * 2,
  )(s, invd, wa, wb, c)


def _tc_post2(sl, sr, invd):
  n, cp = sl.shape[1], sl.shape[2]

  def body(sl_ref, sr_ref, inv_ref, o_ref):
    hl = jnp.maximum((sl_ref[0] + sl_ref[1]) * inv_ref[...], 0.0)
    hr = jnp.maximum((sr_ref[0] + sr_ref[1]) * inv_ref[...], 0.0)
    o_ref[...] = jnp.concatenate([hl, hr], axis=1)

  return pl.pallas_call(
      body,
      grid=(n // _BLK,),
      in_specs=[
          pl.BlockSpec((2, _BLK, cp), lambda i: (0, i, 0)),
          pl.BlockSpec((2, _BLK, cp), lambda i: (0, i, 0)),
          pl.BlockSpec((_BLK, 1), lambda i: (i, 0)),
      ],
      out_specs=pl.BlockSpec((_BLK, 2 * cp), lambda i: (i, 0)),
      out_shape=jax.ShapeDtypeStruct((n, 2 * cp), jnp.float32),
  )(sl, sr, invd)


def _tc_mid(pool, w1a, w1b, b1, w2, b2, wd1, bd1, wd2, bd2, wa0, wb0, c0):
  g4 = pool.shape[2]
  co0 = wa0.shape[1]

  def body(p_ref, w1a_r, w1b_r, b1_r, w2_r, b2_r, wd1_r, bd1_r, wd2_r,
           bd2_r, wa0_r, wb0_r, c0_r, z_ref, a0_ref, b0_ref):
    def step(t, carry):
      su, cn, mx = carry
      return (su + p_ref[t, 0], cn + p_ref[t, 1],
              jnp.maximum(mx, p_ref[t, 2]))
    su, cn, mx = lax.fori_loop(
        0, _NW, step,
        (jnp.zeros((g4, 16), jnp.float32), jnp.zeros((g4, 16), jnp.float32),
         jnp.full((g4, 16), -jnp.inf, jnp.float32)))
    xm = su / jnp.maximum(cn, 1.0)
    g = jnp.maximum(
        jnp.dot(xm, w1a_r[...], preferred_element_type=jnp.float32)
        + jnp.dot(mx, w1b_r[...], preferred_element_type=jnp.float32)
        + b1_r[...], 0.0)
    z = jnp.maximum(
        jnp.dot(g, w2_r[...], preferred_element_type=jnp.float32)
        + b2_r[...], 0.0)
    z_ref[...] = z
    y = jnp.maximum(
        jnp.dot(z, wd1_r[...], preferred_element_type=jnp.float32)
        + bd1_r[...], 0.0)
    y = jnp.maximum(
        jnp.dot(y, wd2_r[...], preferred_element_type=jnp.float32)
        + bd2_r[...], 0.0)
    a0_ref[...] = jnp.dot(y, wa0_r[...],
                          preferred_element_type=jnp.float32) + c0_r[...]
    b0_ref[...] = jnp.dot(y, wb0_r[...], preferred_element_type=jnp.float32)

  return pl.pallas_call(
      body,
      out_shape=[jax.ShapeDtypeStruct((g4, 16), jnp.float32),
                 jax.ShapeDtypeStruct((g4, co0), jnp.float32),
                 jax.ShapeDtypeStruct((g4, co0), jnp.float32)],
  )(pool, w1a, w1b, b1, w2, b2, wd1, bd1, wd2, bd2, wa0, wb0, c0)


def kernel(x, params, edge_index, batch_index):
  n = x.shape[0]
  e = edge_index.shape[1]
  g = 100
  g4 = 104
  npd = _NW * 320
  nch = e // (_NW * _K)
  src3 = edge_index[0].reshape(_NW, nch, _K)
  dst3 = edge_index[1].reshape(_NW, nch, _K)
  xp = jnp.concatenate(
      [x, jnp.zeros((npd - n, x.shape[1]), jnp.float32)], axis=0)

  wa, wb, c = _fold(params["enc0"])
  a, b = _tc_pre0(xp, wa, wb, c)
  s, degp = _edge_pass(a, b, dst3, src3, with_deg=True)

  wa, wb, c = _fold(params["enc1"])
  a, b, invd = _tc_postpre_deg(s, degp.T, wa, wb, c)
  s = _edge_pass(a, b, dst3, src3, with_deg=False)

  wa, wb, c = _fold(params["enc2"])
  a, b = _tc_postpre(s, invd, wa, wb, c)
  s = _edge_pass(a, b, dst3, src3, with_deg=False)

  bp = jnp.concatenate(
      [batch_index, jnp.full((npd - n,), g, jnp.int32)], axis=0)
  pool = _pool_pass(s, invd[:, 0].reshape(_NW, 1, 320),
                    bp.reshape(_NW, 1, 320), g4)

  fc1, fc2 = params["fc1"], params["fc2"]
  dfc1, dfc2 = params["dfc1"], params["dfc2"]
  wa0, wb0, c0 = _fold(params["dec0"])
  nf = fc1["W"].shape[0] // 2
  z4, a0s, b0s = _tc_mid(
      pool, fc1["W"][:nf], fc1["W"][nf:], fc1["b"].reshape(1, -1),
      fc2["W"], fc2["b"].reshape(1, -1),
      dfc1["W"], dfc1["b"].reshape(1, -1),
      dfc2["W"], dfc2["b"].reshape(1, -1),
      wa0, wb0, c0)
  z = z4[:g]

  a = jnp.repeat(a0s[:g], n // g, axis=0)
  b = jnp.repeat(b0s[:g], n // g, axis=0)
  a = jnp.concatenate(
      [a, jnp.zeros((npd - n, a.shape[1]), jnp.float32)], axis=0)
  b = jnp.concatenate(
      [b, jnp.zeros((npd - n, b.shape[1]), jnp.float32)], axis=0)
  s = _edge_pass(a, b, dst3, src3, with_deg=False)

  wa, wb, c = _fold(params["dec1"])
  a, b = _tc_postpre(s, invd, wa, wb, c)
  s = _edge_pass(a, b, dst3, src3, with_deg=False)

  wa, wb, c = _fold(params["dec2"])
  a, b = _tc_postpre(s, invd, wa, wb, c)
  sl, sr = _edge_pass(a[:, :64], b[:, :64], dst3, src3, with_deg=False,
                      ab2=(a[:, 64:], b[:, 64:]))
  xd = _tc_post2(sl, sr, invd)[:n]
  return xd, z
